# baseline (speedup 1.0000x reference)
import functools

import jax
import jax.numpy as jnp
from jax import lax
from jax.experimental import pallas as pl
from jax.experimental.pallas import tpu as pltpu
from jax.experimental.pallas import tpu_sc as plsc

_S = 2048
_D = 1024
_F = 2048
_E = 8
_TM = 256
_T = _S * 2 // _TM + _E
_NPAD = _T * _TM
_NW = 32
_L = 16


def _router_body(w_ref, a1_ref, a2_ref, ind_ref, g1_ref, g2_ref):
    ww = w_ref[...]
    lane = lax.broadcasted_iota(jnp.int32, ww.shape, 1)
    m1 = jnp.max(ww, axis=-1, keepdims=True)
    a1 = jnp.argmax(ww, axis=-1)[:, None]
    masked = jnp.where(lane == a1, -jnp.inf, ww)
    m2 = jnp.max(masked, axis=-1, keepdims=True)
    a2 = jnp.argmax(masked, axis=-1)[:, None]
    denom = m1 + m2
    a1_ref[...] = a1
    a2_ref[...] = a2
    ind_ref[...] = ((lane == a1) | (lane == a2)).astype(jnp.int32)
    g1_ref[...] = m1 / denom
    g2_ref[...] = m2 / denom


def _router(weights):
    return pl.pallas_call(
        _router_body,
        out_shape=(
            jax.ShapeDtypeStruct((_S, 1), jnp.int32),
            jax.ShapeDtypeStruct((_S, 1), jnp.int32),
            jax.ShapeDtypeStruct((_S, _E), jnp.int32),
            jax.ShapeDtypeStruct((_S, 1), jnp.float32),
            jax.ShapeDtypeStruct((_S, 1), jnp.float32),
        ),
    )(weights)


def _dispatch(xi, pos_all):
    mesh = plsc.VectorSubcoreMesh(core_axis_name="c", subcore_axis_name="s")
    rows_per_w = _S // (_NW // 2)

    @functools.partial(
        pl.kernel,
        mesh=mesh,
        out_type=jax.ShapeDtypeStruct((_NPAD, _D // 2), jnp.int32),
        scratch_types=[
            pltpu.VMEM((rows_per_w,), jnp.int32),
            pltpu.VMEM((rows_per_w, _D // 2), jnp.int32),
        ],
    )
    def k(x_hbm, pos_hbm, xs_hbm, idx_v, rows_v):
        wid = lax.axis_index("s") * 2 + lax.axis_index("c")
        pltpu.sync_copy(pos_hbm.at[wid], idx_v)
        tok0 = (wid % (_NW // 2)) * rows_per_w
        pltpu.sync_copy(x_hbm.at[pl.ds(tok0, rows_per_w)], rows_v)
        pltpu.sync_copy(rows_v, xs_hbm.at[idx_v])

    return k(xi, pos_all)


def _gmm_body(te_ref, act_ref, xs_ref, w1_ref, v1_ref, w2_ref, ys_ref):
    i = pl.program_id(0)

    @pl.when(act_ref[i] == 1)
    def _():
        x = xs_ref[...]
        x1 = lax.dot_general(
            x, w1_ref[0], (((1,), (1,)), ((), ())),
            preferred_element_type=jnp.float32)
        x2 = lax.dot_general(
            x, v1_ref[0], (((1,), (1,)), ((), ())),
            preferred_element_type=jnp.float32)
        act = (x1 * lax.logistic(x1) * x2).astype(jnp.bfloat16)
        ys_ref[...] = jnp.dot(act, w2_ref[0],
                              preferred_element_type=jnp.float32)


def _gmm(tile_expert, tile_active, xs, w1r, v1r, w2r):
    grid_spec = pltpu.PrefetchScalarGridSpec(
        num_scalar_prefetch=2,
        grid=(_T,),
        in_specs=[
            pl.BlockSpec((_TM, _D), lambda i, te, act: (i, 0)),
            pl.BlockSpec((1, _F, _D), lambda i, te, act: (te[i], 0, 0)),
            pl.BlockSpec((1, _F, _D), lambda i, te, act: (te[i], 0, 0)),
            pl.BlockSpec((1, _F, _D), lambda i, te, act: (te[i], 0, 0)),
        ],
        out_specs=pl.BlockSpec((_TM, _D), lambda i, te, act: (i, 0)),
    )
    return pl.pallas_call(
        _gmm_body,
        grid_spec=grid_spec,
        out_shape=jax.ShapeDtypeStruct((_NPAD, _D), jnp.float32),
    )(tile_expert, tile_active, xs, w1r, v1r, w2r)


def _combine(ys, pos_c, g1e, g2e):
    mesh = plsc.VectorSubcoreMesh(core_axis_name="c", subcore_axis_name="s")
    tok_per_w = _S // _NW
    half = tok_per_w // 2

    @functools.partial(
        pl.kernel,
        mesh=mesh,
        out_type=jax.ShapeDtypeStruct((_S, _D), jnp.float32),
        scratch_types=[
            pltpu.VMEM((half,), jnp.int32),
            pltpu.VMEM((half, _D), jnp.float32),
            pltpu.VMEM((half, _D), jnp.float32),
            pltpu.VMEM((half, _L), jnp.float32),
        ],
    )
    def k(ys_hbm, pos_hbm, g1_hbm, g2_hbm, out_hbm, idx_v, ybuf, obuf, gbuf):
        wid = lax.axis_index("s") * 2 + lax.axis_index("c")
        for h in range(2):
            base = wid * tok_per_w + h * half
            pltpu.sync_copy(pos_hbm.at[0, pl.ds(base, half)], idx_v)
            pltpu.sync_copy(ys_hbm.at[idx_v], ybuf)
            pltpu.sync_copy(g1_hbm.at[pl.ds(base, half)], gbuf)

            @pl.loop(0, half)
            def _(r):
                gv = gbuf[r, :]
                for c in range(_D // _L):
                    sl = pl.ds(c * _L, _L)
                    obuf[r, sl] = gv * ybuf[r, sl]

            pltpu.sync_copy(pos_hbm.at[1, pl.ds(base, half)], idx_v)
            pltpu.sync_copy(ys_hbm.at[idx_v], ybuf)
            pltpu.sync_copy(g2_hbm.at[pl.ds(base, half)], gbuf)

            @pl.loop(0, half)
            def _(r):
                gv = gbuf[r, :]
                for c in range(_D // _L):
                    sl = pl.ds(c * _L, _L)
                    obuf[r, sl] = obuf[r, sl] + gv * ybuf[r, sl]

            pltpu.sync_copy(obuf, out_hbm.at[pl.ds(base, half)])

    return k(ys, pos_c, g1e, g2e)


def kernel(hidden_states, router_w, w1, v1, w2):
    x = hidden_states.reshape(_S, _D)
    logits = jnp.matmul(x.astype(jnp.float32), router_w)
    weights = jax.nn.softmax(logits.astype(jnp.float32), axis=-1)

    a1, a2, ind, g1, g2 = _router(weights)
    a1 = a1[:, 0]
    a2 = a2[:, 0]

    csum = jnp.cumsum(ind, axis=0)
    cnt_before = csum - ind
    counts = csum[-1]
    tiles_e = (counts + _TM - 1) // _TM
    tile_start = jnp.concatenate(
        [jnp.zeros((1,), jnp.int32), jnp.cumsum(tiles_e)])
    start = tile_start[:_E] * _TM
    pos1 = start[a1] + jnp.take_along_axis(cnt_before, a1[:, None], 1)[:, 0]
    pos2 = start[a2] + jnp.take_along_axis(cnt_before, a2[:, None], 1)[:, 0]

    used_tiles = tile_start[_E]
    ti = jnp.arange(_T, dtype=jnp.int32)
    tile_expert = jnp.minimum(
        (ti[:, None] >= tile_start[None, 1:]).sum(1), _E - 1).astype(jnp.int32)
    tile_active = (ti < used_tiles).astype(jnp.int32)

    pos_all = jnp.concatenate([pos1, pos2]).reshape(_NW, _S // (_NW // 2))
    pos_c = jnp.stack([pos1, pos2]).astype(jnp.int32)
    g1e = jnp.broadcast_to(g1, (_S, _L))
    g2e = jnp.broadcast_to(g2, (_S, _L))

    xb = x.astype(jnp.bfloat16)
    w1r = w1.reshape(_E, _F, _D).astype(jnp.bfloat16)
    v1r = v1.reshape(_E, _F, _D).astype(jnp.bfloat16)
    w2r = w2.reshape(_E, _F, _D).astype(jnp.bfloat16)

    xi = lax.bitcast_convert_type(
        xb.reshape(_S, _D // 2, 2), jnp.int32)
    xsi = _dispatch(xi, pos_all.astype(jnp.int32))
    xs = lax.bitcast_convert_type(
        xsi, jnp.bfloat16).reshape(_NPAD, _D)
    ys = _gmm(tile_expert, tile_active, xs, w1r, v1r, w2r)
    out = _combine(ys, pos_c, g1e, g2e)

    return (out.reshape(hidden_states.shape),
            weights.reshape(hidden_states.shape[0], _S, _E))
